# Initial kernel scaffold; baseline (speedup 1.0000x reference)
import jax
import jax.numpy as jnp
from jax.experimental import pallas as pl


def kernel(x, feat_cache, hit_idx, hit_mask, edge_index0, edge_index1, n_id, emb_cache_table, emb_cache, push_batch_id, push_global_id, pull_batch_id, pull_global_id, W_l0, W_r0, b0, W_l1, W_r1, b1):
    raise NotImplementedError("write your pallas kernel here")


# speedup vs baseline: 2.7243x; 2.7243x over previous
import functools

import jax
import jax.numpy as jnp
from jax import lax
from jax.experimental import pallas as pl
from jax.experimental.pallas import tpu as pltpu
from jax.experimental.pallas import tpu_sc as plsc

N_SRC0 = 50000
N_DST0 = 10000
N_DST1 = 2048
E0 = 320000
E1 = 65536
D = 128
D_OUT = 47
FEAT_ROWS = 200000
EMB_ROWS = 100000
GLOBAL_N = 200000
N_HIT = 25000
N_MISS = 25000
N_PUSH = 2048
N_PULL = 2048

NC = 2
NS = 16
NW = NC * NS

f32 = jnp.float32
i32 = jnp.int32

HP = 28672
A_CH = 112
A_K = 8
H_ROWS = N_SRC0 + 64

B_K = 80
E0P = NW * B_K * 128
ACC0_ROWS = 10240
B_STRIPE = ACC0_ROWS // NS

H1_ROWS = N_DST0 + 48
F_K = E1 // (NW * 128)
F_STRIPE = N_DST1 // NS

_MESH = plsc.VectorSubcoreMesh(
    core_axis_name="c", subcore_axis_name="s", num_cores=NC, num_subcores=NS)


def _wid():
    return lax.axis_index("s") * NC + lax.axis_index("c")


def _zero_2d(ref, rows, lanes):
    per = lanes // 16

    def body(i, _):
        r = i // per
        cc = (i % per) * 16
        ref[r, pl.ds(cc, 16)] = jnp.zeros((16,), ref.dtype)
        return 0

    lax.fori_loop(0, rows * per, body, 0)


def _fill_ones(ref, rows):
    def body(r, _):
        ref[r, :] = jnp.ones((16,), f32)
        return 0

    lax.fori_loop(0, rows, body, 0)


def _slot_rows(tab_hbm, gid_hbm, out_hbm, jb, gid_v, ridx_v, grows_v, sem):
    pltpu.sync_copy(gid_hbm.at[pl.ds(jb, 64)], gid_v)
    for k in range(4):
        g16 = gid_v[pl.ds(k * 16, 16)]
        ridx_v[pl.ds(k * 16, 16)] = lax.shift_right_logical(g16, 7)
    pltpu.async_copy(tab_hbm.at[ridx_v], grows_v, sem).wait()
    pltpu.sync_copy(grows_v, out_hbm.at[pl.ds(jb, 64)])


@functools.partial(
    pl.kernel,
    out_type=(
        jax.ShapeDtypeStruct((H_ROWS, D), f32),
        jax.ShapeDtypeStruct((N_PUSH, 128), i32),
        jax.ShapeDtypeStruct((N_PULL, 128), i32),
    ),
    mesh=_MESH,
    compiler_params=pltpu.CompilerParams(needs_layout_passes=False),
    scratch_types=(
        pltpu.VMEM((A_K, A_CH), i32),
        pltpu.VMEM((A_K, A_CH), i32),
        pltpu.VMEM((A_CH, D), f32),
        pltpu.VMEM((64,), i32),
        pltpu.VMEM((64,), i32),
        pltpu.VMEM((64, 128), i32),
        pltpu.SemaphoreType.DMA,
    ),
)
def _stage_a(feat_hbm, hidx_hbm, hpos_hbm, x_hbm, mpos_hbm, tab_hbm,
             pgid_hbm, lgid_hbm, h_out, prow_out, lrow_out,
             idx_v, pos_v, rows_v, gid_v, ridx_v, grows_v, sem):
    w = _wid()
    rb = w * A_K
    pltpu.sync_copy(hidx_hbm.at[pl.ds(rb, A_K)], idx_v)
    pltpu.sync_copy(hpos_hbm.at[pl.ds(rb, A_K)], pos_v)
    for k in range(A_K):
        pltpu.async_copy(feat_hbm.at[idx_v.at[k]], rows_v, sem).wait()
        pltpu.async_copy(rows_v, h_out.at[pos_v.at[k]], sem).wait()
    pltpu.sync_copy(mpos_hbm.at[pl.ds(rb, A_K)], pos_v)
    for k in range(A_K):
        pltpu.sync_copy(x_hbm.at[pl.ds(w * (A_K * A_CH) + k * A_CH, A_CH)],
                        rows_v)
        pltpu.async_copy(rows_v, h_out.at[pos_v.at[k]], sem).wait()
    jb = w * 64
    _slot_rows(tab_hbm, pgid_hbm, prow_out, jb, gid_v, ridx_v, grows_v, sem)
    _slot_rows(tab_hbm, lgid_hbm, lrow_out, jb, gid_v, ridx_v, grows_v, sem)


def _agg_kernel(n_chunks, acc_rows, stripe, table_hbm, s2d_hbm, d2d_hbm,
                acc_out, hist_out, sidx_v, didx_v, rows_v, hist_v,
                acc_sh, sem):
    hr = acc_rows // 128
    c = lax.axis_index("c")
    s = lax.axis_index("s")
    w = _wid()
    _zero_2d(rows_v, 128, D)
    _zero_2d(hist_v, hr, D)
    base = s * stripe
    full = stripe // 128
    for t in range(full):
        pltpu.sync_copy(rows_v, acc_sh.at[pl.ds(base + t * 128, 128)])
    plsc.subcore_barrier()
    pltpu.sync_copy(s2d_hbm.at[pl.ds(w * n_chunks, n_chunks)], sidx_v)
    pltpu.sync_copy(d2d_hbm.at[pl.ds(w * n_chunks, n_chunks)], didx_v)

    def body(k, _):
        pltpu.async_copy(table_hbm.at[sidx_v.at[k]], rows_v, sem).wait()
        pltpu.sync_copy(rows_v, acc_sh.at[didx_v.at[k]], add=True)
        return 0

    lax.fori_loop(0, n_chunks, body, 0)
    ones16 = jnp.ones((16,), f32)

    def cbody(kk, _):
        k = kk // 8
        l = kk % 8
        d16 = didx_v[k, pl.ds(l * 16, 16)]
        r16 = lax.shift_right_logical(d16, 7)
        l16 = jnp.bitwise_and(d16, 127)
        plsc.addupdate_scatter(hist_v, [r16, l16], ones16)
        return 0

    lax.fori_loop(0, n_chunks * 8, cbody, 0)
    pltpu.sync_copy(hist_v, hist_out.at[pl.ds(w * hr, hr)])
    plsc.subcore_barrier()
    for t in range(full):
        pltpu.sync_copy(acc_sh.at[pl.ds(base + t * 128, 128)],
                        acc_out.at[c, pl.ds(base + t * 128, 128)])


_stage_b = functools.partial(
    pl.kernel,
    out_type=(
        jax.ShapeDtypeStruct((NC, ACC0_ROWS, D), f32),
        jax.ShapeDtypeStruct((NW * (ACC0_ROWS // 128), 128), f32),
    ),
    mesh=_MESH,
    compiler_params=pltpu.CompilerParams(needs_layout_passes=False),
    scratch_types=(
        pltpu.VMEM((B_K, 128), i32),
        pltpu.VMEM((B_K, 128), i32),
        pltpu.VMEM((128, D), f32),
        pltpu.VMEM((ACC0_ROWS // 128, D), f32),
        pltpu.VMEM_SHARED((ACC0_ROWS, D), f32),
        pltpu.SemaphoreType.DMA,
    ),
)(functools.partial(_agg_kernel, B_K, ACC0_ROWS, B_STRIPE))

_stage_f = functools.partial(
    pl.kernel,
    out_type=(
        jax.ShapeDtypeStruct((NC, N_DST1, D), f32),
        jax.ShapeDtypeStruct((NW * (N_DST1 // 128), 128), f32),
    ),
    mesh=_MESH,
    compiler_params=pltpu.CompilerParams(needs_layout_passes=False),
    scratch_types=(
        pltpu.VMEM((F_K, 128), i32),
        pltpu.VMEM((F_K, 128), i32),
        pltpu.VMEM((128, D), f32),
        pltpu.VMEM((N_DST1 // 128, D), f32),
        pltpu.VMEM_SHARED((N_DST1, D), f32),
        pltpu.SemaphoreType.DMA,
    ),
)(functools.partial(_agg_kernel, F_K, N_DST1, F_STRIPE))


def _stage_c_body(acc0, acc1, cnt, ht, wl, wr, b, out):
    c = jnp.sum(cnt[...], axis=0).reshape(-1, 1)
    mean = (acc0[...] + acc1[...]) / jnp.maximum(c, 1.0)
    res = (jnp.dot(mean, wl[...], preferred_element_type=f32)
           + jnp.dot(ht[...], wr[...], preferred_element_type=f32)
           + b[...])
    out[...] = jnp.maximum(res, 0.0)


def _stage_c(acc, cnt, h, W_l0, W_r0, b0):
    blk = 1024
    grid = ACC0_ROWS // blk
    return pl.pallas_call(
        _stage_c_body,
        grid=(grid,),
        in_specs=[
            pl.BlockSpec((blk, D), lambda i: (i, 0)),
            pl.BlockSpec((blk, D), lambda i: (i, 0)),
            pl.BlockSpec((NW, blk), lambda i: (0, i)),
            pl.BlockSpec((blk, D), lambda i: (i, 0)),
            pl.BlockSpec((D, D), lambda i: (0, 0)),
            pl.BlockSpec((D, D), lambda i: (0, 0)),
            pl.BlockSpec((1, D), lambda i: (0, 0)),
        ],
        out_specs=pl.BlockSpec((blk, D), lambda i: (i, 0)),
        out_shape=jax.ShapeDtypeStruct((ACC0_ROWS, D), f32),
    )(acc[0], acc[1], cnt, h[:ACC0_ROWS], W_l0, W_r0, b0.reshape(1, D))


def _stage_d_body(prow, pgid_c, pbid_c, lrow, lgid_b, lbid_c, lbid_r,
                  sc1, sc2, pbx, spull_o):
    j0 = pl.program_id(0) * 128
    lane_p = lax.broadcasted_iota(i32, (N_PUSH, 128), 1)
    spush_c = jnp.sum(
        jnp.where(lane_p == jnp.bitwise_and(pgid_c[...], 127), prow[...], 0),
        axis=1).reshape(N_PUSH, 1)
    lane_l = lax.broadcasted_iota(i32, (128, 128), 1)
    spull_r = jnp.sum(
        jnp.where(lane_l == jnp.bitwise_and(lgid_b[...], 127), lrow[...], 0),
        axis=1).reshape(1, 128)
    ii = lax.broadcasted_iota(i32, (N_PUSH, 128), 0)
    eq = spush_c == spull_r
    win = jnp.max(jnp.where(eq, ii * 16384 + pbid_c[...], -1), axis=0,
                  keepdims=True)
    eq2 = lbid_c[...] == lbid_r[...]
    last = jnp.max(jnp.where(eq2, ii, -1), axis=0, keepdims=True)
    jlane = lax.broadcasted_iota(i32, (1, 128), 1) + j0
    is_last = last == jlane
    tile = jlane >> 7
    hit = win >= 0
    sc1[...] = jnp.where(is_last, lbid_r[...], N_DST0 + tile)
    sc2[...] = jnp.where(jnp.logical_and(is_last, hit), lbid_r[...],
                         N_DST0 + 16 + tile)
    pbx[...] = jnp.where(hit, jnp.bitwise_and(win, 16383), 0)
    spull_o[...] = spull_r


def _stage_d(prow, push_gid, pbid_push, lrow, pull_gid, pull_bid):
    return pl.pallas_call(
        _stage_d_body,
        grid=(N_PULL // 128,),
        in_specs=[
            pl.BlockSpec((N_PUSH, 128), lambda i: (0, 0)),
            pl.BlockSpec((N_PUSH, 1), lambda i: (0, 0)),
            pl.BlockSpec((N_PUSH, 1), lambda i: (0, 0)),
            pl.BlockSpec((128, 128), lambda i: (i, 0)),
            pl.BlockSpec((128, 1), lambda i: (i, 0)),
            pl.BlockSpec((N_PULL, 1), lambda i: (0, 0)),
            pl.BlockSpec((1, 128), lambda i: (0, i)),
        ],
        out_specs=[
            pl.BlockSpec((1, 128), lambda i: (0, i)),
            pl.BlockSpec((1, 128), lambda i: (0, i)),
            pl.BlockSpec((1, 128), lambda i: (0, i)),
            pl.BlockSpec((1, 128), lambda i: (0, i)),
        ],
        out_shape=[
            jax.ShapeDtypeStruct((1, N_PULL), i32),
            jax.ShapeDtypeStruct((1, N_PULL), i32),
            jax.ShapeDtypeStruct((1, N_PULL), i32),
            jax.ShapeDtypeStruct((1, N_PULL), i32),
        ],
    )(prow, push_gid.reshape(N_PUSH, 1), pbid_push.reshape(N_PUSH, 1),
      lrow, pull_gid.reshape(N_PULL, 1), pull_bid.reshape(N_PULL, 1),
      pull_bid.reshape(1, N_PULL))


@functools.partial(
    pl.kernel,
    out_type=jax.ShapeDtypeStruct((H1_ROWS, D), f32),
    mesh=_MESH,
    compiler_params=pltpu.CompilerParams(needs_layout_passes=False),
    scratch_types=(
        pltpu.VMEM((320, D), f32),
        pltpu.VMEM((128,), i32),
        pltpu.VMEM((128,), i32),
        pltpu.VMEM((128,), i32),
        pltpu.VMEM((128,), i32),
        pltpu.VMEM((128, D), f32),
        pltpu.VMEM((128, D), f32),
        pltpu.SemaphoreType.DMA,
    ),
)
def _stage_e(h0_hbm, emb_hbm, spull_hbm, sc1_hbm, sc2_hbm, pbx_hbm, h1_out,
             cp_v, spull_v, idx1_v, idx2_v, pbx_v, vcache_v, vpush_v, sem):
    c = lax.axis_index("c")
    s = lax.axis_index("s")

    @pl.when(c == 0)
    def _():
        b = jnp.minimum(s * 632, N_DST0 - 632)
        pltpu.sync_copy(h0_hbm.at[pl.ds(b, 320)], cp_v)
        pltpu.sync_copy(cp_v, h1_out.at[pl.ds(b, 320)])
        pltpu.sync_copy(h0_hbm.at[pl.ds(b + 320, 312)],
                        cp_v.at[pl.ds(0, 312)])
        pltpu.sync_copy(cp_v.at[pl.ds(0, 312)],
                        h1_out.at[pl.ds(b + 320, 312)])
        jb = s * 128
        pltpu.sync_copy(spull_hbm.at[pl.ds(jb, 128)], spull_v)
        pltpu.sync_copy(sc1_hbm.at[pl.ds(jb, 128)], idx1_v)
        pltpu.sync_copy(sc2_hbm.at[pl.ds(jb, 128)], idx2_v)
        pltpu.sync_copy(pbx_hbm.at[pl.ds(jb, 128)], pbx_v)
        pltpu.async_copy(emb_hbm.at[spull_v], vcache_v, sem).wait()
        pltpu.async_copy(h0_hbm.at[pbx_v], vpush_v, sem).wait()
        plsc.subcore_barrier()
        pltpu.async_copy(vcache_v, h1_out.at[idx1_v], sem).wait()
        plsc.subcore_barrier()
        pltpu.async_copy(vpush_v, h1_out.at[idx2_v], sem).wait()


def _stage_g_body(acc0, acc1, cnt, ht, wl, wr, b, out):
    c = jnp.sum(cnt[...], axis=0).reshape(-1, 1)
    mean = (acc0[...] + acc1[...]) / jnp.maximum(c, 1.0)
    logits = (jnp.dot(mean, wl[...], preferred_element_type=f32)
              + jnp.dot(ht[...], wr[...], preferred_element_type=f32)
              + b[...])[:, :D_OUT]
    m = jnp.max(logits, axis=-1, keepdims=True)
    z = logits - m
    lse = jnp.log(jnp.sum(jnp.exp(z), axis=-1, keepdims=True))
    out[...] = z - lse


def _stage_g(acc, cnt, h1, W_l1, W_r1, b1):
    blk = 512
    wl = jnp.zeros((D, D), f32).at[:, :D_OUT].set(W_l1)
    wr = jnp.zeros((D, D), f32).at[:, :D_OUT].set(W_r1)
    bb = jnp.zeros((1, D), f32).at[0, :D_OUT].set(b1)
    return pl.pallas_call(
        _stage_g_body,
        grid=(N_DST1 // blk,),
        in_specs=[
            pl.BlockSpec((blk, D), lambda i: (i, 0)),
            pl.BlockSpec((blk, D), lambda i: (i, 0)),
            pl.BlockSpec((NW, blk), lambda i: (0, i)),
            pl.BlockSpec((blk, D), lambda i: (i, 0)),
            pl.BlockSpec((D, D), lambda i: (0, 0)),
            pl.BlockSpec((D, D), lambda i: (0, 0)),
            pl.BlockSpec((1, D), lambda i: (0, 0)),
        ],
        out_specs=pl.BlockSpec((blk, D_OUT), lambda i: (i, 0)),
        out_shape=jax.ShapeDtypeStruct((N_DST1, D_OUT), f32),
    )(acc[0], acc[1], cnt, h1[:N_DST1], wl, wr, bb)


def kernel(x, feat_cache, hit_idx, hit_mask, edge_index0, edge_index1, n_id,
           emb_cache_table, emb_cache, push_batch_id, push_global_id,
           pull_batch_id, pull_global_id, W_l0, W_r0, b0, W_l1, W_r1, b1):
    del n_id
    hit_pos = jnp.nonzero(hit_mask, size=N_HIT)[0].astype(i32)
    miss_pos = jnp.nonzero(jnp.logical_not(hit_mask), size=N_MISS)[0].astype(i32)

    hit_idx_p = jnp.zeros((HP,), i32).at[:N_HIT].set(hit_idx.astype(i32))
    hit_pos_p = jnp.full((HP,), N_SRC0, i32).at[:N_HIT].set(hit_pos)
    miss_pos_p = jnp.full((HP,), N_SRC0 + 1, i32).at[:N_MISS].set(miss_pos)
    x_p = jnp.zeros((HP, D), f32).at[:N_MISS].set(x)
    tabr = jnp.zeros((200064,), i32).at[:GLOBAL_N].set(
        emb_cache_table.astype(i32)).reshape(-1, 128)

    h, prow, lrow = _stage_a(
        feat_cache, hit_idx_p.reshape(-1, A_CH), hit_pos_p.reshape(-1, A_CH),
        x_p, miss_pos_p.reshape(-1, A_CH), tabr,
        push_global_id.astype(i32), pull_global_id.astype(i32))

    src0 = jnp.zeros((E0P,), i32).at[:E0].set(edge_index0[0].astype(i32))
    dst0 = jnp.full((E0P,), N_DST0, i32).at[:E0].set(edge_index0[1].astype(i32))
    acc0, hist0 = _stage_b(h, src0.reshape(-1, 128), dst0.reshape(-1, 128))

    h0 = _stage_c(acc0, hist0.reshape(NW, ACC0_ROWS), h, W_l0, W_r0, b0)

    sc1, sc2, pbx, spull = _stage_d(
        prow, push_global_id.astype(i32), push_batch_id.astype(i32),
        lrow, pull_global_id.astype(i32), pull_batch_id.astype(i32))

    h1 = _stage_e(h0, emb_cache, spull.reshape(-1), sc1.reshape(-1),
                  sc2.reshape(-1), pbx.reshape(-1))

    src1 = edge_index1[0].astype(i32)
    dst1 = edge_index1[1].astype(i32)
    acc1, hist1 = _stage_f(h1, src1.reshape(-1, 128), dst1.reshape(-1, 128))

    return _stage_g(acc1, hist1.reshape(NW, N_DST1), h1, W_l1, W_r1, b1)
